# baseline (speedup 1.0000x reference)
import functools

import jax
import jax.numpy as jnp
from jax import lax
from jax.experimental import pallas as pl
from jax.experimental.pallas import tpu as pltpu
from jax.experimental.pallas import tpu_sc as plsc

N = 10000
E = 320000
D = 128
CW = 8
NC = 2
NS = 16
NW = NC * NS
K = 128
EW = E // NW
NP = 10240
NR = NP // NS
CHP = NP // K
TRASH = NP - 1


def _sc_accumulate():
    mesh = plsc.VectorSubcoreMesh(core_axis_name="c", subcore_axis_name="s")

    @functools.partial(
        pl.kernel,
        out_type=(
            jax.ShapeDtypeStruct((NC, NP, D), jnp.float32),
            jax.ShapeDtypeStruct((NC, NP, CW), jnp.float32),
        ),
        mesh=mesh,
        scratch_types=[
            pltpu.VMEM((4, K), jnp.int32),
            pltpu.VMEM((4, K), jnp.int32),
            pltpu.VMEM((K, D), jnp.float32),
            pltpu.VMEM((K, D), jnp.float32),
            pltpu.VMEM((K, CW), jnp.float32),
            pltpu.SemaphoreType.DMA,
            pltpu.SemaphoreType.DMA,
            pltpu.SemaphoreType.DMA,
            pltpu.SemaphoreType.DMA,
            pltpu.SemaphoreType.DMA,
            pltpu.SemaphoreType.DMA,
            pltpu.VMEM_SHARED((NP, D), jnp.float32),
            pltpu.VMEM_SHARED((NP, CW), jnp.float32),
        ],
        compiler_params=pltpu.CompilerParams(use_tc_tiling_on_sc=False),
    )
    def sc_fn(x_hbm, edges_hbm, zsum_hbm, zcnt_hbm, ones_hbm,
              osum_hbm, ocnt_hbm,
              srcb, dstb, rows0, rows1, ones_v,
              g0, g1, gi0, gi1, gi2, gi3, acc, cnt):
        c = lax.axis_index("c")
        s = lax.axis_index("s")
        wid = s * NC + c
        gis = [gi0, gi1, gi2, gi3]

        def idx_fire(jj, slot):
            pltpu.async_copy(edges_hbm.at[0, wid, jj], srcb.at[slot], gis[slot])
            pltpu.async_copy(edges_hbm.at[1, wid, jj], dstb.at[slot], gis[slot])

        def idx_wait(slot):
            d = pltpu.make_async_copy(
                edges_hbm.at[0, wid, 0], srcb.at[slot], gis[slot])
            d.wait()
            d2 = pltpu.make_async_copy(
                edges_hbm.at[1, wid, 0], dstb.at[slot], gis[slot])
            d2.wait()

        def gather(slot, buf, sem):
            pltpu.async_copy(x_hbm.at[srcb.at[slot]], buf, sem)

        def gwait(buf, sem):
            pltpu.make_async_copy(x_hbm.at[srcb.at[0]], buf, sem).wait()

        def scatter(buf, slot):
            pltpu.sync_copy(buf, acc.at[dstb.at[slot]], add=True)
            pltpu.sync_copy(ones_v, cnt.at[dstb.at[slot]], add=True)

        pltpu.sync_copy(zsum_hbm, acc.at[pl.ds(s * NR, NR)])
        pltpu.sync_copy(zcnt_hbm, cnt.at[pl.ds(s * NR, NR)])
        pltpu.sync_copy(ones_hbm, ones_v)
        for m in range(4):
            idx_fire(m, m)
        plsc.subcore_barrier()

        idx_wait(0)
        gather(0, rows0, g0)
        idx_wait(1)
        gather(1, rows1, g1)

        def body(k, carry):
            j = 4 * k
            gwait(rows0, g0)
            scatter(rows0, 0)
            idx_fire(lax.rem(j + 4, CHP), 0)
            idx_wait(2)
            gather(2, rows0, g0)
            gwait(rows1, g1)
            scatter(rows1, 1)
            idx_fire(lax.rem(j + 5, CHP), 1)
            idx_wait(3)
            gather(3, rows1, g1)
            gwait(rows0, g0)
            scatter(rows0, 2)
            idx_fire(lax.rem(j + 6, CHP), 2)
            idx_wait(0)
            gather(0, rows0, g0)
            gwait(rows1, g1)
            scatter(rows1, 3)
            idx_fire(lax.rem(j + 7, CHP), 3)
            idx_wait(1)
            gather(1, rows1, g1)
            return carry

        lax.fori_loop(0, CHP // 4, body, 0)
        gwait(rows0, g0)
        gwait(rows1, g1)
        idx_wait(2)
        idx_wait(3)
        plsc.subcore_barrier()

        pltpu.sync_copy(acc.at[pl.ds(s * NR, NR)],
                        osum_hbm.at[c, pl.ds(s * NR, NR)])
        pltpu.sync_copy(cnt.at[pl.ds(s * NR, NR)],
                        ocnt_hbm.at[c, pl.ds(s * NR, NR)])

    return sc_fn


def _tc_self(x, W_r, b_l):
    BN = 2000

    def body(x_ref, wr_ref, bl_ref, o_ref):
        o_ref[...] = (
            jnp.dot(x_ref[...], wr_ref[...], preferred_element_type=jnp.float32)
            + bl_ref[...]
        )

    return pl.pallas_call(
        body,
        grid=(N // BN,),
        in_specs=[
            pl.BlockSpec((BN, D), lambda i: (i, 0)),
            pl.BlockSpec((D, D), lambda i: (0, 0)),
            pl.BlockSpec((1, D), lambda i: (0, 0)),
        ],
        out_specs=pl.BlockSpec((BN, D), lambda i: (i, 0)),
        out_shape=jax.ShapeDtypeStruct((N, D), jnp.float32),
    )(x, W_r, b_l.reshape(1, D))


def _tc_finish(psum, pcnt, selfterm, W_l):
    BN = 2000

    def body(p_ref, c_ref, s_ref, wl_ref, o_ref):
        summed = p_ref[0] + p_ref[1]
        cnt = c_ref[0][:, 0:1] + c_ref[1][:, 0:1]
        mean = summed / jnp.maximum(cnt, 1.0)
        o_ref[...] = (
            jnp.dot(mean, wl_ref[...], preferred_element_type=jnp.float32)
            + s_ref[...]
        )

    return pl.pallas_call(
        body,
        grid=(N // BN,),
        in_specs=[
            pl.BlockSpec((NC, BN, D), lambda i: (0, i, 0)),
            pl.BlockSpec((NC, BN, CW), lambda i: (0, i, 0)),
            pl.BlockSpec((BN, D), lambda i: (i, 0)),
            pl.BlockSpec((D, D), lambda i: (0, 0)),
        ],
        out_specs=pl.BlockSpec((BN, D), lambda i: (i, 0)),
        out_shape=jax.ShapeDtypeStruct((N, D), jnp.float32),
    )(psum, pcnt, selfterm, W_l)


def kernel(x, edge_index, W_l, b_l, W_r):
    e2 = edge_index.reshape(2, NW, EW)
    pad = jnp.broadcast_to(
        jnp.array([[0], [TRASH]], jnp.int32)[:, None, :],
        (2, NW, CHP * K - EW),
    )
    edges = jnp.concatenate([e2, pad], axis=2).reshape(2, NW, CHP, K)
    zsum = jnp.zeros((NR, D), jnp.float32)
    zcnt = jnp.zeros((NR, CW), jnp.float32)
    ones = jnp.ones((K, CW), jnp.float32)
    selfterm = _tc_self(x, W_r, b_l)
    psum, pcnt = _sc_accumulate()(x, edges, zsum, zcnt, ones)
    return _tc_finish(psum, pcnt, selfterm, W_l)


# speedup vs baseline: 2.6041x; 2.6041x over previous
import functools

import jax
import jax.numpy as jnp
from jax import lax
from jax.experimental import pallas as pl
from jax.experimental.pallas import tpu as pltpu
from jax.experimental.pallas import tpu_sc as plsc

N = 10000
E = 320000
D = 128
CW = 8
NC = 2
NS = 16
NW = NC * NS
K = 80
EK = E // K
CH = EK // NW
NP = 10112
NR = NP // NS


def _sc_accumulate():
    mesh = plsc.VectorSubcoreMesh(core_axis_name="c", subcore_axis_name="s")

    @functools.partial(
        pl.kernel,
        out_type=(
            jax.ShapeDtypeStruct((NC, NP, D), jnp.float32),
            jax.ShapeDtypeStruct((NC, NP, CW), jnp.float32),
        ),
        mesh=mesh,
        scratch_types=[
            pltpu.VMEM((CH, K), jnp.int32),
            pltpu.VMEM((CH, K), jnp.int32),
            pltpu.VMEM((K, D), jnp.float32),
            pltpu.VMEM((K, D), jnp.float32),
            pltpu.VMEM((K, CW), jnp.float32),
            pltpu.SemaphoreType.DMA,
            pltpu.SemaphoreType.DMA,
            pltpu.SemaphoreType.DMA,
            pltpu.VMEM_SHARED((NP, D), jnp.float32),
            pltpu.VMEM_SHARED((NP, CW), jnp.float32),
        ],
        compiler_params=pltpu.CompilerParams(use_tc_tiling_on_sc=False),
    )
    def sc_fn(x_hbm, edges_hbm, zsum_hbm, zcnt_hbm, ones_hbm,
              osum_hbm, ocnt_hbm,
              src_v, dst_v, rows0, rows1, ones_v, sem0, sem1, cs, acc, cnt):
        c = lax.axis_index("c")
        s = lax.axis_index("s")
        wid = s * NC + c

        pltpu.sync_copy(zsum_hbm, acc.at[pl.ds(s * NR, NR)])
        pltpu.sync_copy(zcnt_hbm, cnt.at[pl.ds(s * NR, NR)])
        pltpu.sync_copy(ones_hbm, ones_v)
        pltpu.sync_copy(edges_hbm.at[0, wid], src_v)
        pltpu.sync_copy(edges_hbm.at[1, wid], dst_v)
        plsc.subcore_barrier()

        pltpu.async_copy(x_hbm.at[src_v.at[0]], rows0, sem0)

        def body(k, carry):
            j = 2 * k
            pltpu.make_async_copy(x_hbm.at[src_v.at[j]], rows0, sem0).wait()
            pltpu.async_copy(x_hbm.at[src_v.at[j + 1]], rows1, sem1)
            pltpu.sync_copy(rows0, acc.at[dst_v.at[j]], add=True)
            pltpu.async_copy(ones_v, cnt.at[dst_v.at[j]], cs, add=True)
            pltpu.async_copy(x_hbm.at[src_v.at[j + 2]], rows0, sem0)
            pltpu.make_async_copy(x_hbm.at[src_v.at[j + 1]], rows1, sem1).wait()
            pltpu.sync_copy(rows1, acc.at[dst_v.at[j + 1]], add=True)
            pltpu.async_copy(ones_v, cnt.at[dst_v.at[j + 1]], cs, add=True)
            pltpu.make_async_copy(ones_v, cnt.at[dst_v.at[j]], cs).wait()
            pltpu.make_async_copy(ones_v, cnt.at[dst_v.at[j + 1]], cs).wait()
            return carry

        lax.fori_loop(0, (CH - 1) // 2, body, 0)
        pltpu.make_async_copy(x_hbm.at[src_v.at[CH - 1]], rows0, sem0).wait()
        pltpu.sync_copy(rows0, acc.at[dst_v.at[CH - 1]], add=True)
        pltpu.sync_copy(ones_v, cnt.at[dst_v.at[CH - 1]], add=True)
        plsc.subcore_barrier()

        pltpu.sync_copy(acc.at[pl.ds(s * NR, NR)],
                        osum_hbm.at[c, pl.ds(s * NR, NR)])
        pltpu.sync_copy(cnt.at[pl.ds(s * NR, NR)],
                        ocnt_hbm.at[c, pl.ds(s * NR, NR)])

    return sc_fn


def _tc_self(x, W_r, b_l):
    BN = 2000

    def body(x_ref, wr_ref, bl_ref, o_ref):
        o_ref[...] = (
            jnp.dot(x_ref[...], wr_ref[...], preferred_element_type=jnp.float32)
            + bl_ref[...]
        )

    return pl.pallas_call(
        body,
        grid=(N // BN,),
        in_specs=[
            pl.BlockSpec((BN, D), lambda i: (i, 0)),
            pl.BlockSpec((D, D), lambda i: (0, 0)),
            pl.BlockSpec((1, D), lambda i: (0, 0)),
        ],
        out_specs=pl.BlockSpec((BN, D), lambda i: (i, 0)),
        out_shape=jax.ShapeDtypeStruct((N, D), jnp.float32),
    )(x, W_r, b_l.reshape(1, D))


def _tc_finish(psum, pcnt, selfterm, W_l):
    BN = 2000

    def body(p_ref, c_ref, s_ref, wl_ref, o_ref):
        summed = p_ref[0] + p_ref[1]
        cnt = c_ref[0][:, 0:1] + c_ref[1][:, 0:1]
        mean = summed / jnp.maximum(cnt, 1.0)
        o_ref[...] = (
            jnp.dot(mean, wl_ref[...], preferred_element_type=jnp.float32)
            + s_ref[...]
        )

    return pl.pallas_call(
        body,
        grid=(N // BN,),
        in_specs=[
            pl.BlockSpec((NC, BN, D), lambda i: (0, i, 0)),
            pl.BlockSpec((NC, BN, CW), lambda i: (0, i, 0)),
            pl.BlockSpec((BN, D), lambda i: (i, 0)),
            pl.BlockSpec((D, D), lambda i: (0, 0)),
        ],
        out_specs=pl.BlockSpec((BN, D), lambda i: (i, 0)),
        out_shape=jax.ShapeDtypeStruct((N, D), jnp.float32),
    )(psum, pcnt, selfterm, W_l)


def kernel(x, edge_index, W_l, b_l, W_r):
    edges = edge_index.reshape(2, NW, CH, K)
    zsum = jnp.zeros((NR, D), jnp.float32)
    zcnt = jnp.zeros((NR, CW), jnp.float32)
    ones = jnp.ones((K, CW), jnp.float32)
    selfterm = _tc_self(x, W_r, b_l)
    psum, pcnt = _sc_accumulate()(x, edges, zsum, zcnt, ones)
    return _tc_finish(psum, pcnt, selfterm, W_l)
